# baseline (speedup 1.0000x reference)
import functools
import math

import jax
import jax.numpy as jnp
from jax import lax
from jax.experimental import pallas as pl
from jax.experimental.pallas import tpu as pltpu
from jax.experimental.pallas import tpu_sc as plsc

_SIZE = 16384
_SMOOTH = 0.1
_CONF = 0.9
_S = _SMOOTH / (_SIZE - 2)


def _ls_kernel(ts_ref, t_ref, x_ref, o_ref, *, n_blocks):
    i = pl.program_id(0)
    xb = x_ref[...]
    tcol = t_ref[0]
    br = xb.shape[0]
    rowsum = jnp.sum(xb, axis=1, keepdims=True)
    x0 = xb[:, 0:1]
    k_const = _SMOOTH * jnp.log(_S) + _CONF * jnp.log(_CONF)
    del k_const
    contrib = jnp.where(tcol != 0, -_S * rowsum + _S * x0, 0.0)

    lane = jax.lax.broadcasted_iota(jnp.int32, (1, 128), 1)
    acc = jnp.zeros((1, 128), jnp.float32)
    for r in range(br):
        t = ts_ref[0, r, 0]
        v = x_ref[pl.ds(r, 1), pl.ds((t >> 7) * 128, 128)]
        acc += jnp.where((lane == (t & 127)) & (t != 0), v, 0.0)

    total = (jnp.sum(contrib) + (_S - _CONF) * jnp.sum(acc)).reshape(1, 1)

    @pl.when(i == 0)
    def _():
        o_ref[...] = jnp.zeros_like(o_ref)

    o_ref[...] += total


def _sc_kconst(t_hbm, out_hbm, idx_v, acc_v, *, bpw, lanes, ncores):
    wid = lax.axis_index("s") * ncores + lax.axis_index("c")
    base = wid * bpw
    k_const = _SMOOTH * math.log(_S) + _CONF * math.log(_CONF)
    pltpu.sync_copy(t_hbm.at[pl.ds(base, bpw)], idx_v)
    acc = jnp.zeros((lanes,), jnp.float32)
    for k in range(bpw // lanes):
        t16 = idx_v[pl.ds(k * lanes, lanes)]
        acc = acc + jnp.where(t16 != 0, k_const, 0.0)
    acc_v[...] = acc
    pltpu.sync_copy(acc_v, out_hbm.at[pl.ds(wid * lanes, lanes)])


def _kconst_partials(target):
    import functools as _ft
    info = plsc.get_sparse_core_info()
    nc, ns, lanes = info.num_cores, info.num_subcores, info.num_lanes
    nw = nc * ns
    bpw = target.shape[0] // nw
    mesh = plsc.VectorSubcoreMesh(core_axis_name="c", subcore_axis_name="s")
    return pl.kernel(
        _ft.partial(_sc_kconst, bpw=bpw, lanes=lanes, ncores=nc),
        out_type=jax.ShapeDtypeStruct((nw * lanes,), jnp.float32),
        mesh=mesh,
        scratch_types=[
            pltpu.VMEM((bpw,), jnp.int32),
            pltpu.VMEM((lanes,), jnp.float32),
        ],
    )(target)


def kernel(x, target):
    n, c = x.shape
    br = 128
    n_blocks = n // br
    tr = target.reshape(n_blocks, br, 1)
    out = pl.pallas_call(
        functools.partial(_ls_kernel, n_blocks=n_blocks),
        grid=(n_blocks,),
        in_specs=[
            pl.BlockSpec((1, br, 1), lambda i: (i, 0, 0),
                         memory_space=pltpu.SMEM),
            pl.BlockSpec((1, br, 1), lambda i: (i, 0, 0)),
            pl.BlockSpec((br, c), lambda i: (i, 0)),
        ],
        out_specs=pl.BlockSpec((1, 1), lambda i: (0, 0)),
        out_shape=jax.ShapeDtypeStruct((1, 1), jnp.float32),
    )(tr, tr, x)
    kpart = _kconst_partials(target)
    return out[0, 0] + jnp.sum(kpart)


# speedup vs baseline: 1.2304x; 1.2304x over previous
import functools

import jax
import jax.numpy as jnp
from jax.experimental import pallas as pl
from jax.experimental.pallas import tpu as pltpu

_SIZE = 16384
_SMOOTH = 0.1
_CONF = 0.9
_S = _SMOOTH / (_SIZE - 2)


def _ls_kernel(ts_ref, t_ref, x_ref, o_ref, *, n_blocks):
    i = pl.program_id(0)
    xb = x_ref[...]
    tcol = t_ref[0]
    br = xb.shape[0]
    rowsum = jnp.sum(xb, axis=1, keepdims=True)
    x0 = xb[:, 0:1]
    k_const = _SMOOTH * jnp.log(_S) + _CONF * jnp.log(_CONF)
    contrib = jnp.where(tcol != 0, k_const - _S * rowsum + _S * x0, 0.0)

    lane = jax.lax.broadcasted_iota(jnp.int32, (1, 128), 1)
    acc = jnp.zeros((1, 128), jnp.float32)
    for r in range(br):
        t = ts_ref[0, r, 0]
        v = x_ref[pl.ds(r, 1), pl.ds((t >> 7) * 128, 128)]
        acc += jnp.where((lane == (t & 127)) & (t != 0), v, 0.0)

    total = (jnp.sum(contrib) + (_S - _CONF) * jnp.sum(acc)).reshape(1, 1)

    @pl.when(i == 0)
    def _():
        o_ref[...] = jnp.zeros_like(o_ref)

    o_ref[...] += total


def kernel(x, target):
    n, c = x.shape
    br = 128
    n_blocks = n // br
    tr = target.reshape(n_blocks, br, 1)
    out = pl.pallas_call(
        functools.partial(_ls_kernel, n_blocks=n_blocks),
        grid=(n_blocks,),
        in_specs=[
            pl.BlockSpec((1, br, 1), lambda i: (i, 0, 0),
                         memory_space=pltpu.SMEM),
            pl.BlockSpec((1, br, 1), lambda i: (i, 0, 0)),
            pl.BlockSpec((br, c), lambda i: (i, 0)),
        ],
        out_specs=pl.BlockSpec((1, 1), lambda i: (0, 0)),
        out_shape=jax.ShapeDtypeStruct((1, 1), jnp.float32),
    )(tr, tr, x)
    return out[0, 0]
